# baseline (speedup 1.0000x reference)
import functools

import jax
import jax.numpy as jnp
import numpy as np
from jax import lax
from jax.experimental import pallas as pl
from jax.experimental.pallas import tpu as pltpu
from jax.experimental.pallas import tpu_sc as plsc

_B, _L, _D = 1024, 200, 64
_NW = 32
_SEQ_PER_W = _B // _NW
_C0, _C1 = 104, 96


def _pe01_table():
    position = jnp.arange(0, _L, dtype=jnp.float32)[:, None]
    div_term = jnp.exp(
        jnp.arange(0, _D, 2, dtype=jnp.float32) * -(np.log(10000.0) / _D))
    pe = jnp.zeros((_L, _D), dtype=jnp.float32)
    pe = pe.at[:, 0::2].set(jnp.sin(position * div_term))
    pe = pe.at[:, 1::2].set(jnp.cos(position * div_term))
    return 0.01 * pe


_mesh = plsc.VectorSubcoreMesh(core_axis_name="c", subcore_axis_name="s")


@functools.partial(
    pl.kernel,
    mesh=_mesh,
    compiler_params=pltpu.CompilerParams(use_tc_tiling_on_sc=False),
    out_type=[jax.ShapeDtypeStruct((_B * _L, _D), jnp.float32)] * 3,
    scratch_types=[
        pltpu.VMEM((_L,), jnp.int32),
        pltpu.VMEM((_L, _D), jnp.float32),
        pltpu.VMEM((_L, _D), jnp.float32),
        pltpu.VMEM((_L, _D), jnp.float32),
        pltpu.VMEM((_L, _D), jnp.float32),
        pltpu.VMEM((_L, _D), jnp.float32),
        pltpu.SemaphoreType.DMA,
    ],
)
def _emb_kernel(idx_hbm, table_hbm, posw_hbm, pe01_hbm,
                x_hbm, x1_hbm, x2_hbm,
                idx_v, tok_v, x_v, x2_v, posw_v, pe01_v, sem):
    wid = lax.axis_index("s") * 2 + lax.axis_index("c")
    pltpu.sync_copy(posw_hbm, posw_v)
    pltpu.sync_copy(pe01_hbm, pe01_v)

    def seq_body(s, carry):
        base = (wid * _SEQ_PER_W + s) * _L
        pltpu.sync_copy(idx_hbm.at[pl.ds(base, _L)], idx_v)
        cp0 = pltpu.async_copy(
            table_hbm.at[idx_v.at[pl.ds(0, _C0)]], tok_v.at[pl.ds(0, _C0)], sem)
        cp1 = pltpu.async_copy(
            table_hbm.at[idx_v.at[pl.ds(_C0, _C1)]], tok_v.at[pl.ds(_C0, _C1)],
            sem)
        cp0.wait()
        cp1.wait()

        def row_body(r, rc):
            for c in range(_D // 16):
                sl = pl.ds(c * 16, 16)
                t = tok_v[r, sl]
                x_v[r, sl] = t + posw_v[r, sl]
                x2_v[r, sl] = t + pe01_v[r, sl]
            return rc

        lax.fori_loop(0, _L, row_body, 0)
        pltpu.sync_copy(tok_v, x1_hbm.at[pl.ds(base, _L)])
        pltpu.sync_copy(x_v, x_hbm.at[pl.ds(base, _L)])
        pltpu.sync_copy(x2_v, x2_hbm.at[pl.ds(base, _L)])
        return carry

    lax.fori_loop(0, _SEQ_PER_W, seq_body, 0)


def kernel(batch_seqs, token_table, pos_weight):
    idx = batch_seqs.reshape(-1).astype(jnp.int32)
    pe01 = _pe01_table()
    x, x1, x2 = _emb_kernel(idx, token_table, pos_weight, pe01)
    shape = (_B, _L, _D)
    return x.reshape(shape), x1.reshape(shape), x2.reshape(shape)


# speedup vs baseline: 1.5389x; 1.5389x over previous
import functools

import jax
import jax.numpy as jnp
import numpy as np
from jax import lax
from jax.experimental import pallas as pl
from jax.experimental.pallas import tpu as pltpu
from jax.experimental.pallas import tpu_sc as plsc

_B, _L, _D = 1024, 200, 64
_NW = 32
_SEQ_PER_W = _B // _NW
_C0, _C1 = 104, 96


def _pe01_table():
    position = jnp.arange(0, _L, dtype=jnp.float32)[:, None]
    div_term = jnp.exp(
        jnp.arange(0, _D, 2, dtype=jnp.float32) * -(np.log(10000.0) / _D))
    pe = jnp.zeros((_L, _D), dtype=jnp.float32)
    pe = pe.at[:, 0::2].set(jnp.sin(position * div_term))
    pe = pe.at[:, 1::2].set(jnp.cos(position * div_term))
    return 0.01 * pe


_mesh = plsc.VectorSubcoreMesh(core_axis_name="c", subcore_axis_name="s")


@functools.partial(
    pl.kernel,
    mesh=_mesh,
    compiler_params=pltpu.CompilerParams(use_tc_tiling_on_sc=False),
    out_type=[jax.ShapeDtypeStruct((_B * _L, _D), jnp.float32)] * 3,
    scratch_types=[
        pltpu.VMEM((_SEQ_PER_W * _L,), jnp.int32),
        pltpu.VMEM((_L, _D), jnp.float32),
        pltpu.VMEM((_L, _D), jnp.float32),
        pltpu.VMEM((_L, _D), jnp.float32),
        pltpu.VMEM((_L, _D), jnp.float32),
        pltpu.VMEM((_L, _D), jnp.float32),
        pltpu.VMEM((_L, _D), jnp.float32),
        pltpu.VMEM((_L, _D), jnp.float32),
        pltpu.VMEM((_L, _D), jnp.float32),
        pltpu.SemaphoreType.DMA,
        pltpu.SemaphoreType.DMA,
        pltpu.SemaphoreType.DMA,
        pltpu.SemaphoreType.DMA,
        pltpu.SemaphoreType.DMA,
        pltpu.SemaphoreType.DMA,
        pltpu.SemaphoreType.DMA,
        pltpu.SemaphoreType.DMA,
    ],
)
def _emb_kernel(idx_hbm, table_hbm, posw_hbm, pe01_hbm,
                x_hbm, x1_hbm, x2_hbm,
                idx_all, tok0, tok1, xa0, xa1, xb0, xb1, posw_v, pe01_v,
                g0, g1, s1a, s1b, sxa, sxb, s2a, s2b):
    wid = lax.axis_index("s") * 2 + lax.axis_index("c")
    base_all = wid * (_SEQ_PER_W * _L)
    pltpu.sync_copy(idx_hbm.at[pl.ds(base_all, _SEQ_PER_W * _L)], idx_all)
    pltpu.sync_copy(posw_hbm, posw_v)
    pltpu.sync_copy(pe01_hbm, pe01_v)

    toks = (tok0, tok1)
    xs = (xa0, xa1)
    x2s = (xb0, xb1)
    gsem = (g0, g1)
    s1sem = (s1a, s1b)
    xsem = (sxa, sxb)
    x2sem = (s2a, s2b)

    def gather_copies(s, p):
        off = s * _L
        c0 = pltpu.make_async_copy(
            table_hbm.at[idx_all.at[pl.ds(off, _C0)]],
            toks[p].at[pl.ds(0, _C0)], gsem[p])
        c1 = pltpu.make_async_copy(
            table_hbm.at[idx_all.at[pl.ds(off + _C0, _C1)]],
            toks[p].at[pl.ds(_C0, _C1)], gsem[p])
        return c0, c1

    def issue_gather(s, p):
        for c in gather_copies(s, p):
            c.start()

    def wait_gather(s, p):
        for c in gather_copies(s, p):
            c.wait()

    issue_gather(0, 0)
    issue_gather(1, 1)

    def outer(i, carry):
        for p in range(2):
            s = i * 2 + p
            gbase = base_all + s * _L
            wait_gather(s, p)
            cp1 = pltpu.make_async_copy(
                toks[p], x1_hbm.at[pl.ds(gbase, _L)], s1sem[p])
            cp1.start()

            @pl.when(i > 0)
            def _wait_prev_writes():
                pltpu.make_async_copy(
                    xs[p], x_hbm.at[pl.ds(gbase, _L)], xsem[p]).wait()
                pltpu.make_async_copy(
                    x2s[p], x2_hbm.at[pl.ds(gbase, _L)], x2sem[p]).wait()

            def row_body(r, rc):
                for c in range(_D // 16):
                    sl = pl.ds(c * 16, 16)
                    t = toks[p][r, sl]
                    xs[p][r, sl] = t + posw_v[r, sl]
                    x2s[p][r, sl] = t + pe01_v[r, sl]
                return rc

            lax.fori_loop(0, _L, row_body, 0)
            pltpu.make_async_copy(
                xs[p], x_hbm.at[pl.ds(gbase, _L)], xsem[p]).start()
            pltpu.make_async_copy(
                x2s[p], x2_hbm.at[pl.ds(gbase, _L)], x2sem[p]).start()
            cp1.wait()

            @pl.when(i < (_SEQ_PER_W // 2 - 1))
            def _prefetch_next():
                issue_gather(s + 2, p)

        return carry

    lax.fori_loop(0, _SEQ_PER_W // 2, outer, 0)
    for p in range(2):
        pltpu.make_async_copy(
            xs[p], x_hbm.at[pl.ds(base_all, _L)], xsem[p]).wait()
        pltpu.make_async_copy(
            x2s[p], x2_hbm.at[pl.ds(base_all, _L)], x2sem[p]).wait()


def kernel(batch_seqs, token_table, pos_weight):
    idx = batch_seqs.reshape(-1).astype(jnp.int32)
    pe01 = _pe01_table()
    x, x1, x2 = _emb_kernel(idx, token_table, pos_weight, pe01)
    shape = (_B, _L, _D)
    return x.reshape(shape), x1.reshape(shape), x2.reshape(shape)
